# baseline (speedup 1.0000x reference)
import functools

import jax
import jax.numpy as jnp
from jax import lax
from jax.experimental import pallas as pl
from jax.experimental.pallas import tpu as pltpu
from jax.experimental.pallas import tpu_sc as plsc

_CB = 4
_PITCH = 521
_W = 512


def _table_format(embd_t):
    d, v = embd_t.shape
    n_slabs_full = v // _W
    tail = v - n_slabs_full * _W
    per_w = n_slabs_full // 32
    rem = n_slabs_full - per_w * 32
    mesh = plsc.VectorSubcoreMesh(core_axis_name="c", subcore_axis_name="s")
    nbuf = 2

    scratch = (
        [pltpu.VMEM((d, _W + 1), jnp.float32) for _ in range(nbuf)]
        + [pltpu.VMEM((_W // 4, 128), jnp.float32) for _ in range(nbuf)]
        + [pltpu.SemaphoreType.DMA for _ in range(2 * nbuf)]
    )

    @functools.partial(
        pl.kernel,
        mesh=mesh,
        out_type=jax.ShapeDtypeStruct((v // 4, 128), jnp.float32),
        scratch_types=scratch,
        compiler_params=pltpu.CompilerParams(needs_layout_passes=False),
    )
    def fmt_kernel(src_hbm, dst_hbm, *bufs):
        xb = bufs[:nbuf]
        tp = bufs[nbuf : 2 * nbuf]
        ss = bufs[2 * nbuf : 3 * nbuf]
        so = bufs[3 * nbuf :]
        wid = lax.axis_index("s") * 2 + lax.axis_index("c")

        def slab_col(t):
            return pl.multiple_of((t * 32 + wid) * _W, 128)

        def stage(t, b, w):
            pltpu.async_copy(
                src_hbm.at[:, pl.ds(slab_col(t), w)],
                xb[b].at[:, pl.ds(0, w)],
                ss[b],
            )

        def wait_stage(t, b, w):
            pltpu.make_async_copy(
                src_hbm.at[:, pl.ds(slab_col(t), w)],
                xb[b].at[:, pl.ds(0, w)],
                ss[b],
            ).wait()

        def transpose(b, n_emb, i0=0):
            jv = lax.iota(jnp.int32, 16)
            j0v = jv
            j1v = jv + 16
            zs = jnp.zeros((16,), jnp.int32)

            def tbody(q, carry):
                qv = zs + q
                for half in range(2):
                    rows = j0v if half == 0 else j1v
                    for il in range(4):
                        col = zs + (i0 + q * 4 + il)
                        vals = plsc.load_gather(xb[b], [rows, col])
                        plsc.store_scatter(
                            tp[b], [qv, jv + (il * 32 + half * 16)], vals
                        )
                return carry

            lax.fori_loop(0, n_emb // 4, tbody, 0, unroll=2)

        def out_dma(t, b, n_emb, start):
            row0 = pl.multiple_of(((t * 32 + wid) * _W) // 4, 8)
            cp = pltpu.make_async_copy(
                tp[b].at[pl.ds(0, n_emb // 4)],
                dst_hbm.at[pl.ds(row0, n_emb // 4)],
                so[b],
            )
            cp.start() if start else cp.wait()

        stage(0, 0, _W)

        def group(g, carry):
            for b in range(nbuf):
                t = g * nbuf + b
                bn = (b + 1) % nbuf

                @pl.when(t + 1 < per_w)
                def _():
                    stage(t + 1, bn, _W)

                wait_stage(t, b, _W)

                @pl.when(t >= nbuf)
                def _():
                    out_dma(t - nbuf, b, _W, False)

                transpose(b, _W)
                out_dma(t, b, _W, True)
            return carry

        lax.fori_loop(0, per_w // nbuf, group, 0)
        rem_t = per_w - (per_w // nbuf) * nbuf
        for r in range(rem_t):
            t = (per_w // nbuf) * nbuf + r
            b = t % nbuf
            wait_stage(t, b, _W)
            out_dma(t - nbuf, b, _W, False)
            transpose(b, _W)
            out_dma(t, b, _W, True)
        for t in range(per_w - nbuf, per_w):
            out_dma(t, t % nbuf, _W, False)

        @pl.when(wid == 0)
        def _():
            c0 = n_slabs_full * _W - _W
            pltpu.sync_copy(src_hbm.at[:, pl.ds(c0, _W)], xb[0].at[:, pl.ds(0, _W)])
            transpose(0, _W)
            pltpu.sync_copy(tp[0], dst_hbm.at[pl.ds(c0 // 4, _W // 4)])


    return fmt_kernel(embd_t)


def _emb_lookup(idx2, table, tail_rows, bsz, fld, d):
    tbs = bsz // 128
    n_blocks = fld * tbs
    chunk = 128 * _CB
    per_w = n_blocks // 32 // _CB
    nbuf = 2
    n_tail = tail_rows.shape[0]
    v_main = table.shape[0] - n_tail
    m_rows = fld * (d // 8) * tbs * 8
    mesh = plsc.VectorSubcoreMesh(core_axis_name="c", subcore_axis_name="s")

    scratch = (
        [pltpu.VMEM((chunk,), jnp.int32) for _ in range(nbuf)]
        + [pltpu.VMEM((chunk, d), jnp.float32) for _ in range(nbuf)]
        + [pltpu.VMEM((d, _PITCH), jnp.float32) for _ in range(nbuf)]
        + [pltpu.VMEM((n_tail, d), jnp.float32)]
        + [pltpu.SemaphoreType.DMA for _ in range(3 * nbuf)]
    )

    @functools.partial(
        pl.kernel,
        mesh=mesh,
        out_type=jax.ShapeDtypeStruct((m_rows, 128), jnp.float32),
        scratch_types=scratch,
        compiler_params=pltpu.CompilerParams(
            use_tc_tiling_on_sc=False, needs_layout_passes=False
        ),
    )
    def emb_kernel(idx_hbm, table_hbm, tail_hbm, out2_hbm, *bufs):
        xi = bufs[:nbuf]
        gb = bufs[nbuf : 2 * nbuf]
        segb = bufs[2 * nbuf : 3 * nbuf]
        tw = bufs[3 * nbuf]
        si = bufs[3 * nbuf + 1 : 4 * nbuf + 1]
        sg = bufs[4 * nbuf + 1 : 5 * nbuf + 1]
        so = bufs[5 * nbuf + 1 :]
        wid = lax.axis_index("s") * 2 + lax.axis_index("c")
        b0 = wid * per_w * _CB
        pltpu.sync_copy(tail_hbm, tw)

        def blk(k):
            c = b0 + k * _CB
            f = lax.shift_right_logical(c, 7)
            tb = lax.bitwise_and(c, jnp.int32(127))
            return f, tb

        def idx_off(k):
            f, tb = blk(k)
            return f * bsz + tb * 128

        def stage_idx(k, b):
            pltpu.async_copy(idx_hbm.at[pl.ds(idx_off(k), chunk)], xi[b], si[b])

        def wait_idx(k, b):
            pltpu.make_async_copy(
                idx_hbm.at[pl.ds(idx_off(k), chunk)], xi[b], si[b]
            ).wait()

        def start_gather(b):
            pltpu.async_copy(table_hbm.at[xi[b]], gb[b], sg[b])

        def wait_gather(b):
            pltpu.make_async_copy(table_hbm.at[xi[b]], gb[b], sg[b]).wait()

        def fixup(b):
            jv = lax.iota(jnp.int32, 16)
            zs = jnp.zeros((16,), jnp.int32)

            def fgroup(m, carry):
                x16 = xi[b][pl.ds(m * 16, 16)]
                msk = x16 >= jnp.int32(v_main)

                @pl.when(jnp.any(msk))
                def _():
                    r16 = jv + m * 16
                    t16 = jnp.where(msk, x16 - jnp.int32(v_main), 0)
                    for j in range(d):
                        vals = plsc.load_gather(tw, [t16, zs + j])
                        plsc.store_scatter(gb[b], [r16, zs + j], vals, mask=msk)

                return carry

            lax.fori_loop(0, chunk // 16, fgroup, 0)

        def transpose(b):
            jv = lax.iota(jnp.int32, 16)
            zs = jnp.zeros((16,), jnp.int32)

            def tbody(r, carry):
                col = zs + r
                for h in range(d // 16):
                    vals = gb[b][r, pl.ds(16 * h, 16)]
                    plsc.store_scatter(segb[b], [jv + 16 * h, col], vals)
                return carry

            lax.fori_loop(0, chunk, tbody, 0, unroll=4)

        def out_slabs(k, b, start):
            f, tb = blk(k)
            for tj in range(d // 8):
                for tbl in range(_CB):
                    row0 = ((f * (d // 8) + tj) * tbs + tb + tbl) * 8
                    cp = pltpu.make_async_copy(
                        segb[b].at[pl.ds(tj * 8, 8), pl.ds(tbl * 128, 128)],
                        out2_hbm.at[pl.ds(row0, 8)],
                        so[b],
                    )
                    cp.start() if start else cp.wait()

        for b in range(nbuf):
            stage_idx(b, b)
        wait_idx(0, 0)
        start_gather(0)

        def group(g, carry):
            for b in range(nbuf):
                k = g * nbuf + b
                bn = (b + 1) % nbuf

                @pl.when(k + 1 < per_w)
                def _():
                    wait_idx(k + 1, bn)
                    start_gather(bn)

                @pl.when(k >= nbuf)
                def _():
                    out_slabs(k - nbuf, b, False)

                wait_gather(b)
                fixup(b)
                transpose(b)
                out_slabs(k, b, True)

                @pl.when(k + nbuf < per_w)
                def _():
                    stage_idx(k + nbuf, b)

            return carry

        lax.fori_loop(0, per_w // nbuf, group, 0)
        for k in range(per_w - nbuf, per_w):
            out_slabs(k, k % nbuf, False)

    return emb_kernel(idx2, table, tail_rows)


def kernel(x, embd):
    bsz, fld = x.shape
    v, d = embd.shape
    s4 = _table_format(embd.T)
    table = s4.reshape(v, d)
    tail_rows = embd[v // 128 * 128 :]
    idx2 = x.T.reshape(bsz * fld)
    out2 = _emb_lookup(idx2, table, tail_rows, bsz, fld, d)
    o = out2.reshape(fld, d // 8, bsz // 128, 8, 128)
    o = o.transpose(2, 4, 0, 1, 3)
    return o.reshape(bsz, fld, d)


# speedup vs baseline: 1.4513x; 1.4513x over previous
import functools

import jax
import jax.numpy as jnp
from jax import lax
from jax.experimental import pallas as pl
from jax.experimental.pallas import tpu as pltpu
from jax.experimental.pallas import tpu_sc as plsc

_CB = 4
_PITCH = 521


def _emb_lookup(idx2, embd, bsz, fld, d):
    tbs = bsz // 128
    n_blocks = fld * tbs
    n_workers = 32
    chunk = 128 * _CB
    per_w = n_blocks // n_workers // _CB
    nbuf = 2
    m_rows = fld * (d // 8) * tbs * 8
    mesh = plsc.VectorSubcoreMesh(core_axis_name="c", subcore_axis_name="s")

    scratch = (
        [pltpu.VMEM((chunk,), jnp.int32) for _ in range(nbuf)]
        + [pltpu.VMEM((chunk, d), jnp.float32) for _ in range(nbuf)]
        + [pltpu.VMEM((d, _PITCH), jnp.float32) for _ in range(nbuf)]
        + [pltpu.SemaphoreType.DMA for _ in range(3 * nbuf)]
    )

    @functools.partial(
        pl.kernel,
        mesh=mesh,
        out_type=jax.ShapeDtypeStruct((m_rows, 128), jnp.float32),
        scratch_types=scratch,
        compiler_params=pltpu.CompilerParams(
            use_tc_tiling_on_sc=False, needs_layout_passes=False
        ),
    )
    def emb_kernel(idx_hbm, table_hbm, out2_hbm, *bufs):
        xi = bufs[:nbuf]
        gb = bufs[nbuf : 2 * nbuf]
        segb = bufs[2 * nbuf : 3 * nbuf]
        si = bufs[3 * nbuf : 4 * nbuf]
        sg = bufs[4 * nbuf : 5 * nbuf]
        so = bufs[5 * nbuf :]
        wid = lax.axis_index("s") * 2 + lax.axis_index("c")
        b0 = wid * per_w * _CB

        def blk(k):
            c = b0 + k * _CB
            f = lax.shift_right_logical(c, 7)
            tb = lax.bitwise_and(c, jnp.int32(127))
            return f, tb

        def idx_off(k):
            f, tb = blk(k)
            return f * bsz + tb * 128

        def stage_idx(k, b):
            pltpu.async_copy(idx_hbm.at[pl.ds(idx_off(k), chunk)], xi[b], si[b])

        def wait_idx(k, b):
            pltpu.make_async_copy(
                idx_hbm.at[pl.ds(idx_off(k), chunk)], xi[b], si[b]
            ).wait()

        def start_gather(b):
            pltpu.async_copy(table_hbm.at[xi[b]], gb[b], sg[b])

        def wait_gather(b):
            pltpu.make_async_copy(table_hbm.at[xi[b]], gb[b], sg[b]).wait()

        def transpose(b):
            jv = lax.iota(jnp.int32, 16)
            zs = jnp.zeros((16,), jnp.int32)

            def tbody(r, carry):
                col = zs + r
                for h in range(d // 16):
                    vals = gb[b][r, pl.ds(16 * h, 16)]
                    plsc.store_scatter(segb[b], [jv + 16 * h, col], vals)
                return carry

            lax.fori_loop(0, chunk, tbody, 0, unroll=4)

        def out_slabs(k, b, make_only):
            f, tb = blk(k)
            for tj in range(d // 8):
                for tbl in range(_CB):
                    row0 = ((f * (d // 8) + tj) * tbs + tb + tbl) * 8
                    cp = pltpu.make_async_copy(
                        segb[b].at[pl.ds(tj * 8, 8), pl.ds(tbl * 128, 128)],
                        out2_hbm.at[pl.ds(row0, 8)],
                        so[b],
                    )
                    if make_only:
                        cp.wait()
                    else:
                        cp.start()

        for b in range(nbuf):
            stage_idx(b, b)
        wait_idx(0, 0)
        start_gather(0)

        n_groups = per_w // nbuf

        def group(g, carry):
            for b in range(nbuf):
                k = g * nbuf + b
                bn = (b + 1) % nbuf

                @pl.when(k + 1 < per_w)
                def _():
                    wait_idx(k + 1, bn)
                    start_gather(bn)

                @pl.when(k >= nbuf)
                def _():
                    out_slabs(k - nbuf, b, True)

                wait_gather(b)
                transpose(b)
                out_slabs(k, b, False)

                @pl.when(k + nbuf < per_w)
                def _():
                    stage_idx(k + nbuf, b)

            return carry

        lax.fori_loop(0, n_groups, group, 0)
        for k in range(per_w - nbuf, per_w):
            out_slabs(k, k % nbuf, True)

    return emb_kernel(idx2, embd)


def kernel(x, embd):
    bsz, fld = x.shape
    v, d = embd.shape
    idx2 = x.T.reshape(bsz * fld)
    out2 = _emb_lookup(idx2, embd, bsz, fld, d)
    o = out2.reshape(fld, d // 8, bsz // 128, 8, 128)
    o = o.transpose(2, 4, 0, 1, 3)
    return o.reshape(bsz, fld, d)
